# baseline (speedup 1.0000x reference)
import jax
import jax.numpy as jnp
from jax import lax
from jax.experimental import pallas as pl
from jax.experimental.pallas import tpu as pltpu
from jax.experimental.pallas import tpu_sc as plsc

VOCAB = 1000000
DIM = 64
B = 4096
L = 200

NC = 2
NS = 16
NW = NC * NS

N_ROWS = B * L
ROWS_PER_W = N_ROWS // NW
GRP = 128
N_GRP = ROWS_PER_W // GRP


def _make_kernel():
    mesh = plsc.VectorSubcoreMesh(core_axis_name="c", subcore_axis_name="s")

    @pl.kernel(
        mesh=mesh,
        compiler_params=pltpu.CompilerParams(use_tc_tiling_on_sc=False),
        out_type=jax.ShapeDtypeStruct((NW, N_GRP, GRP, DIM), jnp.float32),
        scratch_types=[
            pltpu.VMEM((N_GRP, GRP), jnp.int32),
            pltpu.VMEM((GRP, DIM), jnp.float32),
            pltpu.SemaphoreType.DMA,
        ],
    )
    def gather_kernel(table_hbm, words_hbm, out_hbm, idx_v, rows_v, gsem):
        wid = lax.axis_index("s") * NC + lax.axis_index("c")
        pltpu.sync_copy(words_hbm.at[wid], idx_v)

        def step(j, carry):
            pltpu.async_copy(table_hbm.at[idx_v.at[j]], rows_v, gsem).wait()
            pltpu.sync_copy(rows_v, out_hbm.at[wid, j])
            return carry

        lax.fori_loop(0, N_GRP, step, 0)

    return gather_kernel


_kernel = _make_kernel()


@jax.jit
def kernel(words, table):
    words_flat = words.reshape(NW, N_GRP, GRP)
    out = _kernel(table, words_flat)
    return out.reshape(B, L, DIM)


# speedup vs baseline: 1.1122x; 1.1122x over previous
import jax
import jax.numpy as jnp
from jax import lax
from jax.experimental import pallas as pl
from jax.experimental.pallas import tpu as pltpu
from jax.experimental.pallas import tpu_sc as plsc

VOCAB = 1000000
DIM = 64
B = 4096
L = 200

NC = 2
NS = 16
NW = NC * NS

N_ROWS = B * L
ROWS_PER_W = N_ROWS // NW
GRP = 128
N_GRP = ROWS_PER_W // GRP
NBUF = 8
LAG = 2
LA = NBUF - LAG


def _make_kernel():
    mesh = plsc.VectorSubcoreMesh(core_axis_name="c", subcore_axis_name="s")

    @pl.kernel(
        mesh=mesh,
        compiler_params=pltpu.CompilerParams(use_tc_tiling_on_sc=False),
        out_type=jax.ShapeDtypeStruct((NW, N_GRP, GRP, DIM), jnp.float32),
        scratch_types=[
            pltpu.VMEM((N_GRP, GRP), jnp.int32),
            pltpu.VMEM((NBUF, GRP, DIM), jnp.float32),
        ]
        + [pltpu.SemaphoreType.DMA] * (2 * NBUF),
    )
    def gather_kernel(table_hbm, words_hbm, out_hbm, idx_v, rows_v, *sems):
        gsem = sems[:NBUF]
        wsem = sems[NBUF:]
        wid = lax.axis_index("s") * NC + lax.axis_index("c")
        pltpu.sync_copy(words_hbm.at[wid], idx_v)

        def fire_gather(t, b):
            pltpu.async_copy(table_hbm.at[idx_v.at[t]], rows_v.at[b], gsem[b])

        def drain_gather(b):
            pltpu.make_async_copy(
                table_hbm.at[idx_v.at[0]], rows_v.at[b], gsem[b]
            ).wait()

        def fire_write(t, b):
            pltpu.async_copy(rows_v.at[b], out_hbm.at[wid, t], wsem[b])

        def drain_write(b):
            pltpu.make_async_copy(
                rows_v.at[b], out_hbm.at[wid, 0], wsem[b]
            ).wait()

        for t in range(LA):
            fire_gather(t, t)
        for t in range(LAG):
            drain_gather(t)
            fire_write(t, t)
            fire_gather(t + LA, t + LA)
        for t in range(LAG, NBUF):
            drain_gather(t)
            fire_write(t, t)
            drain_write(t - LAG)
            fire_gather(t + LA, t - LAG)

        def step(t2, carry):
            for b in range(NBUF):
                t = t2 * NBUF + b
                drain_gather(b)
                fire_write(t, b)
                b2 = (b - LAG) % NBUF
                drain_write(b2)
                fire_gather(t + LA, b2)
            return carry

        lax.fori_loop(1, N_GRP // NBUF - 1, step, 0)

        base = N_GRP - NBUF
        for i in range(LAG):
            drain_gather(i)
            fire_write(base + i, i)
            b2 = (i - LAG) % NBUF
            drain_write(b2)
            fire_gather(base + i + LA, b2)
        for i in range(LAG, NBUF):
            drain_gather(i)
            fire_write(base + i, i)
            drain_write((i - LAG) % NBUF)
        for b in range(NBUF - LAG, NBUF):
            drain_write(b)

    return gather_kernel


_kernel = _make_kernel()


@jax.jit
def kernel(words, table):
    words_flat = words.reshape(NW, N_GRP, GRP)
    out = _kernel(table, words_flat)
    return out.reshape(B, L, DIM)
